# baseline (speedup 1.0000x reference)
import functools

import jax
import jax.numpy as jnp
import numpy as np
from jax.experimental import pallas as pl

NUM_GRAPHS = 256
EMBED = 32
N_LAYERS = 3
EPS = 1e-5
GAMMA = 10.0


def _seg_sum(x, seg, num):
    return jax.ops.segment_sum(x, seg, num_segments=num)


def _rbf_embed(vals, centers, W, b):
    r = jnp.exp(-GAMMA * jnp.square(vals[:, None] - centers[None, :]))
    return r @ W + b


def _mlp(x, p):
    h = jax.nn.relu(x @ p['W1'] + p['b1'])
    return h @ p['W2'] + p['b2']


def _gine(x, edge_index, edge_attr, p, num_nodes):
    msg = jax.nn.relu(x[edge_index[0]] + edge_attr)
    agg = _seg_sum(msg, edge_index[1], num_nodes)
    return _mlp(x + agg, p)


def _layer_norm_graph(x, batch, w, b):
    n = jnp.clip(_seg_sum(jnp.ones((x.shape[0],), x.dtype), batch, NUM_GRAPHS), 1.0)
    denom = n * x.shape[-1]
    mean = _seg_sum(x.sum(-1), batch, NUM_GRAPHS) / denom
    xc = x - mean[batch][:, None]
    var = _seg_sum(jnp.sum(xc * xc, -1), batch, NUM_GRAPHS) / denom
    out = xc / jnp.sqrt(var + EPS)[batch][:, None]
    return out * w + b


def _graph_norm(x, batch, w, b, ms):
    n = jnp.clip(_seg_sum(jnp.ones((x.shape[0],), x.dtype), batch, NUM_GRAPHS), 1.0)[:, None]
    mean = (_seg_sum(x, batch, NUM_GRAPHS) / n)[batch]
    out = x - mean * ms
    var = (_seg_sum(out * out, batch, NUM_GRAPHS) / n)[batch]
    return w * out / jnp.sqrt(var + EPS) + b


def _gin_block(x, edge_index, edge_attr, batch, p, num_nodes):
    out = _gine(x, edge_index, edge_attr, p, num_nodes)
    out = _layer_norm_graph(out, batch, p['ln_w'], p['ln_b'])
    out = _graph_norm(out, batch, p['gn_w'], p['gn_b'], p['gn_ms'])
    out = jax.nn.relu(out)
    return x + out


def _pool_div_kernel(s_ref, c_ref, o_ref):
    o_ref[...] = s_ref[...] / c_ref[...]


def kernel(x, edge_attr, bond_lengths_g, bond_lengths_ex, bond_bond_angles_g,
           bond_bond_angles_ex, params, edge_index, bond_bond_index,
           edge_attr_batch, batch):
    centers_bl = jnp.asarray(np.arange(0, 2, 0.1), jnp.float32)
    centers_ba = jnp.asarray(np.arange(0, np.pi, 0.1), jnp.float32)
    atom_x = x @ params['embed_atom']['W'] + params['embed_atom']['b']
    edge_x = edge_attr @ params['embed_bond']['W'] + params['embed_bond']['b']
    edge_x = edge_x + _rbf_embed(bond_lengths_g, centers_bl, params['bl_fc']['W'], params['bl_fc']['b'])
    edge_x = edge_x + _rbf_embed(bond_lengths_ex, centers_bl, params['bl_fc']['W'], params['bl_fc']['b'])
    angle_x = _rbf_embed(bond_bond_angles_g, centers_ba, params['ba_fc']['W'], params['ba_fc']['b'])
    angle_x = angle_x + _rbf_embed(bond_bond_angles_ex, centers_ba, params['ba_fc']['W'], params['ba_fc']['b'])
    num_nodes = x.shape[0]
    num_edges = edge_attr.shape[0]
    for i in range(N_LAYERS):
        edge_x = _gin_block(edge_x, bond_bond_index, angle_x, edge_attr_batch, params['bond_layers'][i], num_edges)
        atom_x = _gin_block(atom_x, edge_index, edge_x, batch, params['atom_layers'][i], num_nodes)
    counts = jnp.clip(_seg_sum(jnp.ones((num_nodes,), x.dtype), batch, NUM_GRAPHS), 1.0)[:, None]
    sums = _seg_sum(atom_x, batch, NUM_GRAPHS)
    return pl.pallas_call(
        _pool_div_kernel,
        out_shape=jax.ShapeDtypeStruct((NUM_GRAPHS, EMBED), jnp.float32),
    )(sums, jnp.broadcast_to(counts, (NUM_GRAPHS, EMBED)))


# speedup vs baseline: 4.0575x; 4.0575x over previous
import functools

import jax
import jax.numpy as jnp
import numpy as np
from jax import lax
from jax.experimental import pallas as pl
from jax.experimental.pallas import tpu as pltpu

NUM_GRAPHS = 256
EMBED = 32
N_LAYERS = 3
EPS = 1e-5
GAMMA = 10.0

BLK = 4000


def _mlp_stats_body(x_ref, agg_ref, b_ref, w1_ref, b1_ref, w2_ref, b2_ref,
                    y_ref, s_ref):
    h = x_ref[...] + agg_ref[...]
    t = jnp.maximum(h @ w1_ref[...] + b1_ref[...], 0.0)
    y = t @ w2_ref[...] + b2_ref[...]
    y_ref[...] = y
    ids = b_ref[0]
    oh = (ids[:, :, None] == lax.broadcasted_iota(jnp.int32, (1, 1, NUM_GRAPHS), 2))
    oh = oh.reshape(BLK, NUM_GRAPHS).astype(jnp.float32)
    yaug = jnp.concatenate([y, y * y, jnp.ones((BLK, 1), jnp.float32)], axis=1)
    part = lax.dot_general(oh, yaug, (((0,), (0,)), ((), ())),
                           preferred_element_type=jnp.float32)

    @pl.when(pl.program_id(0) == 0)
    def _():
        s_ref[...] = jnp.zeros_like(s_ref)

    s_ref[...] += part


def _mlp_stats(x, agg, batch3, p):
    m = x.shape[0]
    nb = m // BLK
    return pl.pallas_call(
        _mlp_stats_body,
        grid=(nb,),
        in_specs=[
            pl.BlockSpec((BLK, EMBED), lambda i: (i, 0)),
            pl.BlockSpec((BLK, EMBED), lambda i: (i, 0)),
            pl.BlockSpec((1, 1, BLK), lambda i: (i, 0, 0)),
            pl.BlockSpec((EMBED, 2 * EMBED), lambda i: (0, 0)),
            pl.BlockSpec((1, 2 * EMBED), lambda i: (0, 0)),
            pl.BlockSpec((2 * EMBED, EMBED), lambda i: (0, 0)),
            pl.BlockSpec((1, EMBED), lambda i: (0, 0)),
        ],
        out_specs=[
            pl.BlockSpec((BLK, EMBED), lambda i: (i, 0)),
            pl.BlockSpec((NUM_GRAPHS, 2 * EMBED + 1), lambda i: (0, 0)),
        ],
        out_shape=[
            jax.ShapeDtypeStruct((m, EMBED), jnp.float32),
            jax.ShapeDtypeStruct((NUM_GRAPHS, 2 * EMBED + 1), jnp.float32),
        ],
    )(x, agg, batch3, p['W1'], p['b1'][None, :], p['W2'], p['b2'][None, :])


def _rowwise_body(x_ref, y_ref, b_ref, k1_ref, k2_ref, o_ref):
    ids = b_ref[0]
    oh = (ids[:, :, None] == lax.broadcasted_iota(jnp.int32, (1, 1, NUM_GRAPHS), 2))
    oh = oh.reshape(BLK, NUM_GRAPHS).astype(jnp.float32)
    k1r = oh @ k1_ref[...]
    k2r = oh @ k2_ref[...]
    o_ref[...] = x_ref[...] + jnp.maximum(y_ref[...] * k1r + k2r, 0.0)


def _rowwise(x, y, batch3, k1, k2):
    m = x.shape[0]
    nb = m // BLK
    return pl.pallas_call(
        _rowwise_body,
        grid=(nb,),
        in_specs=[
            pl.BlockSpec((BLK, EMBED), lambda i: (i, 0)),
            pl.BlockSpec((BLK, EMBED), lambda i: (i, 0)),
            pl.BlockSpec((1, 1, BLK), lambda i: (i, 0, 0)),
            pl.BlockSpec((NUM_GRAPHS, EMBED), lambda i: (0, 0)),
            pl.BlockSpec((NUM_GRAPHS, EMBED), lambda i: (0, 0)),
        ],
        out_specs=pl.BlockSpec((BLK, EMBED), lambda i: (i, 0)),
        out_shape=jax.ShapeDtypeStruct((m, EMBED), jnp.float32),
    )(x, y, batch3, k1, k2)


def _pool_body(x_ref, b_ref, s_ref):
    ids = b_ref[0]
    oh = (ids[:, :, None] == lax.broadcasted_iota(jnp.int32, (1, 1, NUM_GRAPHS), 2))
    oh = oh.reshape(BLK, NUM_GRAPHS).astype(jnp.float32)
    xaug = jnp.concatenate([x_ref[...], jnp.ones((BLK, 1), jnp.float32)], axis=1)
    part = lax.dot_general(oh, xaug, (((0,), (0,)), ((), ())),
                           preferred_element_type=jnp.float32)

    @pl.when(pl.program_id(0) == 0)
    def _():
        s_ref[...] = jnp.zeros_like(s_ref)

    s_ref[...] += part


def _pool_stats(x, batch3):
    m = x.shape[0]
    nb = m // BLK
    return pl.pallas_call(
        _pool_body,
        grid=(nb,),
        in_specs=[
            pl.BlockSpec((BLK, EMBED), lambda i: (i, 0)),
            pl.BlockSpec((1, 1, BLK), lambda i: (i, 0, 0)),
        ],
        out_specs=pl.BlockSpec((NUM_GRAPHS, EMBED + 1), lambda i: (0, 0)),
        out_shape=jax.ShapeDtypeStruct((NUM_GRAPHS, EMBED + 1), jnp.float32),
    )(x, batch3)


def _norm_coeffs(s, p):
    s1 = s[:, :EMBED]
    s2 = s[:, EMBED:2 * EMBED]
    n = jnp.clip(s[:, 2 * EMBED], 1.0)
    denom = n * EMBED
    mean_ln = s1.sum(-1) / denom
    var_ln = s2.sum(-1) / denom - mean_ln * mean_ln
    inv_ln = 1.0 / jnp.sqrt(var_ln + EPS)
    lnw, lnb = p['ln_w'], p['ln_b']
    za = inv_ln[:, None] * lnw[None, :]
    zc = lnb[None, :] - mean_ln[:, None] * za
    ey = s1 / n[:, None]
    ey2 = s2 / n[:, None]
    ez = za * ey + zc
    ez2 = za * za * ey2 + 2.0 * za * zc * ey + zc * zc
    ms, gnw, gnb = p['gn_ms'], p['gn_w'], p['gn_b']
    var2 = ez2 - 2.0 * ms[None, :] * ez * ez + (ms * ms)[None, :] * ez * ez
    rinv2 = 1.0 / jnp.sqrt(var2 + EPS)
    k1 = za * gnw[None, :] * rinv2
    k2 = (zc - ez * ms[None, :]) * gnw[None, :] * rinv2 + gnb[None, :]
    return k1, k2


def _rbf_embed(vals, centers, W, b):
    r = jnp.exp(-GAMMA * jnp.square(vals[:, None] - centers[None, :]))
    return r @ W + b


def _gin_block(x, src, dst, edge_attr, batch3, p, num_nodes):
    msg = jax.nn.relu(x[src] + edge_attr)
    agg = jax.ops.segment_sum(msg, dst, num_segments=num_nodes)
    y, s = _mlp_stats(x, agg, batch3, p)
    k1, k2 = _norm_coeffs(s, p)
    return _rowwise(x, y, batch3, k1, k2)


def kernel(x, edge_attr, bond_lengths_g, bond_lengths_ex, bond_bond_angles_g,
           bond_bond_angles_ex, params, edge_index, bond_bond_index,
           edge_attr_batch, batch):
    centers_bl = jnp.asarray(np.arange(0, 2, 0.1), jnp.float32)
    centers_ba = jnp.asarray(np.arange(0, np.pi, 0.1), jnp.float32)
    atom_x = x @ params['embed_atom']['W'] + params['embed_atom']['b']
    edge_x = edge_attr @ params['embed_bond']['W'] + params['embed_bond']['b']
    edge_x = edge_x + _rbf_embed(bond_lengths_g, centers_bl, params['bl_fc']['W'], params['bl_fc']['b'])
    edge_x = edge_x + _rbf_embed(bond_lengths_ex, centers_bl, params['bl_fc']['W'], params['bl_fc']['b'])
    angle_x = _rbf_embed(bond_bond_angles_g, centers_ba, params['ba_fc']['W'], params['ba_fc']['b'])
    angle_x = angle_x + _rbf_embed(bond_bond_angles_ex, centers_ba, params['ba_fc']['W'], params['ba_fc']['b'])

    num_nodes = x.shape[0]
    num_edges = edge_attr.shape[0]
    ebatch3 = edge_attr_batch.astype(jnp.int32).reshape(num_edges // BLK, 1, BLK)
    nbatch3 = batch.astype(jnp.int32).reshape(num_nodes // BLK, 1, BLK)

    for i in range(N_LAYERS):
        edge_x = _gin_block(edge_x, bond_bond_index[0], bond_bond_index[1],
                            angle_x, ebatch3, params['bond_layers'][i], num_edges)
        atom_x = _gin_block(atom_x, edge_index[0], edge_index[1],
                            edge_x, nbatch3, params['atom_layers'][i], num_nodes)

    s = _pool_stats(atom_x, nbatch3)
    return s[:, :EMBED] / jnp.clip(s[:, EMBED], 1.0)[:, None]


# speedup vs baseline: 5.0242x; 1.2382x over previous
import functools

import jax
import jax.numpy as jnp
import numpy as np
from jax import lax
from jax.experimental import pallas as pl
from jax.experimental.pallas import tpu as pltpu
from jax.experimental.pallas import tpu_sc as plsc

NUM_GRAPHS = 256
EMBED = 32
PADW = 128
N_LAYERS = 3
EPS = 1e-5
GAMMA = 10.0

BLK = 4000


_SC_MESH = plsc.VectorSubcoreMesh(core_axis_name="c", subcore_axis_name="s")
_SC_C = 400


def _sc_gather(table, idx):
    n = idx.shape[0]
    per_w = n // 32
    nchunk = per_w // _SC_C

    @functools.partial(
        pl.kernel, mesh=_SC_MESH,
        out_type=jax.ShapeDtypeStruct((n, PADW), jnp.float32),
        scratch_types=[
            pltpu.VMEM((_SC_C,), jnp.int32),
            pltpu.VMEM((_SC_C, PADW), jnp.float32),
            pltpu.SemaphoreType.DMA,
        ],
    )
    def k(table_h, idx_h, out_h, idx_v, rows_v, sem):
        wid = lax.axis_index("s") * 2 + lax.axis_index("c")
        base = wid * per_w

        def body(j, carry):
            off = base + j * _SC_C
            pltpu.sync_copy(idx_h.at[pl.ds(off, _SC_C)], idx_v)
            pltpu.async_copy(table_h.at[idx_v], rows_v, sem).wait()
            pltpu.sync_copy(rows_v, out_h.at[pl.ds(off, _SC_C)])
            return carry

        lax.fori_loop(0, nchunk, body, 0)

    return k(table, idx)


def _mlp_stats_body(x_ref, agg_ref, b_ref, w1_ref, b1_ref, w2_ref, b2_ref,
                    y_ref, s_ref):
    h = x_ref[:, :EMBED] + agg_ref[...]
    t = jnp.maximum(h @ w1_ref[...] + b1_ref[...], 0.0)
    y = t @ w2_ref[...] + b2_ref[...]
    y_ref[...] = jnp.concatenate(
        [y, jnp.zeros((BLK, PADW - EMBED), jnp.float32)], axis=1)
    ids = b_ref[0]
    oh = (ids[:, :, None] == lax.broadcasted_iota(jnp.int32, (1, 1, NUM_GRAPHS), 2))
    oh = oh.reshape(BLK, NUM_GRAPHS).astype(jnp.float32)
    yaug = jnp.concatenate([y, y * y, jnp.ones((BLK, 1), jnp.float32)], axis=1)
    part = lax.dot_general(oh, yaug, (((0,), (0,)), ((), ())),
                           preferred_element_type=jnp.float32)

    @pl.when(pl.program_id(0) == 0)
    def _():
        s_ref[...] = jnp.zeros_like(s_ref)

    s_ref[...] += part


def _mlp_stats(x, agg, batch3, p):
    m = x.shape[0]
    nb = m // BLK
    return pl.pallas_call(
        _mlp_stats_body,
        grid=(nb,),
        in_specs=[
            pl.BlockSpec((BLK, PADW), lambda i: (i, 0)),
            pl.BlockSpec((BLK, EMBED), lambda i: (i, 0)),
            pl.BlockSpec((1, 1, BLK), lambda i: (i, 0, 0)),
            pl.BlockSpec((EMBED, 2 * EMBED), lambda i: (0, 0)),
            pl.BlockSpec((1, 2 * EMBED), lambda i: (0, 0)),
            pl.BlockSpec((2 * EMBED, EMBED), lambda i: (0, 0)),
            pl.BlockSpec((1, EMBED), lambda i: (0, 0)),
        ],
        out_specs=[
            pl.BlockSpec((BLK, PADW), lambda i: (i, 0)),
            pl.BlockSpec((NUM_GRAPHS, 2 * EMBED + 1), lambda i: (0, 0)),
        ],
        out_shape=[
            jax.ShapeDtypeStruct((m, PADW), jnp.float32),
            jax.ShapeDtypeStruct((NUM_GRAPHS, 2 * EMBED + 1), jnp.float32),
        ],
    )(x, agg, batch3, p['W1'], p['b1'][None, :], p['W2'], p['b2'][None, :])


def _rowwise_body(x_ref, y_ref, b_ref, k1_ref, k2_ref, o_ref):
    ids = b_ref[0]
    oh = (ids[:, :, None] == lax.broadcasted_iota(jnp.int32, (1, 1, NUM_GRAPHS), 2))
    oh = oh.reshape(BLK, NUM_GRAPHS).astype(jnp.float32)
    k1r = oh @ k1_ref[...]
    k2r = oh @ k2_ref[...]
    o = x_ref[:, :EMBED] + jnp.maximum(y_ref[:, :EMBED] * k1r + k2r, 0.0)
    o_ref[...] = jnp.concatenate(
        [o, jnp.zeros((BLK, PADW - EMBED), jnp.float32)], axis=1)


def _rowwise(x, y, batch3, k1, k2):
    m = x.shape[0]
    nb = m // BLK
    return pl.pallas_call(
        _rowwise_body,
        grid=(nb,),
        in_specs=[
            pl.BlockSpec((BLK, PADW), lambda i: (i, 0)),
            pl.BlockSpec((BLK, PADW), lambda i: (i, 0)),
            pl.BlockSpec((1, 1, BLK), lambda i: (i, 0, 0)),
            pl.BlockSpec((NUM_GRAPHS, EMBED), lambda i: (0, 0)),
            pl.BlockSpec((NUM_GRAPHS, EMBED), lambda i: (0, 0)),
        ],
        out_specs=pl.BlockSpec((BLK, PADW), lambda i: (i, 0)),
        out_shape=jax.ShapeDtypeStruct((m, PADW), jnp.float32),
    )(x, y, batch3, k1, k2)


def _pool_body(x_ref, b_ref, s_ref):
    ids = b_ref[0]
    oh = (ids[:, :, None] == lax.broadcasted_iota(jnp.int32, (1, 1, NUM_GRAPHS), 2))
    oh = oh.reshape(BLK, NUM_GRAPHS).astype(jnp.float32)
    xaug = jnp.concatenate([x_ref[:, :EMBED], jnp.ones((BLK, 1), jnp.float32)],
                           axis=1)
    part = lax.dot_general(oh, xaug, (((0,), (0,)), ((), ())),
                           preferred_element_type=jnp.float32)

    @pl.when(pl.program_id(0) == 0)
    def _():
        s_ref[...] = jnp.zeros_like(s_ref)

    s_ref[...] += part


def _pool_stats(x, batch3):
    m = x.shape[0]
    nb = m // BLK
    return pl.pallas_call(
        _pool_body,
        grid=(nb,),
        in_specs=[
            pl.BlockSpec((BLK, PADW), lambda i: (i, 0)),
            pl.BlockSpec((1, 1, BLK), lambda i: (i, 0, 0)),
        ],
        out_specs=pl.BlockSpec((NUM_GRAPHS, EMBED + 1), lambda i: (0, 0)),
        out_shape=jax.ShapeDtypeStruct((NUM_GRAPHS, EMBED + 1), jnp.float32),
    )(x, batch3)


def _norm_coeffs(s, p):
    s1 = s[:, :EMBED]
    s2 = s[:, EMBED:2 * EMBED]
    n = jnp.clip(s[:, 2 * EMBED], 1.0)
    denom = n * EMBED
    mean_ln = s1.sum(-1) / denom
    var_ln = s2.sum(-1) / denom - mean_ln * mean_ln
    inv_ln = 1.0 / jnp.sqrt(var_ln + EPS)
    lnw, lnb = p['ln_w'], p['ln_b']
    za = inv_ln[:, None] * lnw[None, :]
    zc = lnb[None, :] - mean_ln[:, None] * za
    ey = s1 / n[:, None]
    ey2 = s2 / n[:, None]
    ez = za * ey + zc
    ez2 = za * za * ey2 + 2.0 * za * zc * ey + zc * zc
    ms, gnw, gnb = p['gn_ms'], p['gn_w'], p['gn_b']
    var2 = ez2 - 2.0 * ms[None, :] * ez * ez + (ms * ms)[None, :] * ez * ez
    rinv2 = 1.0 / jnp.sqrt(var2 + EPS)
    k1 = za * gnw[None, :] * rinv2
    k2 = (zc - ez * ms[None, :]) * gnw[None, :] * rinv2 + gnb[None, :]
    return k1, k2


def _rbf_embed(vals, centers, W, b):
    r = jnp.exp(-GAMMA * jnp.square(vals[:, None] - centers[None, :]))
    return r @ W + b


def _gin_block(x, src, dst, edge_attr, batch3, p, num_nodes):
    g = _sc_gather(x, src)
    msg = jax.nn.relu(g[:, :EMBED] + edge_attr)
    agg = jax.ops.segment_sum(msg, dst, num_segments=num_nodes)
    y, s = _mlp_stats(x, agg, batch3, p)
    k1, k2 = _norm_coeffs(s, p)
    return _rowwise(x, y, batch3, k1, k2)


def _pad128(x):
    return jnp.pad(x, ((0, 0), (0, PADW - x.shape[1])))


def kernel(x, edge_attr, bond_lengths_g, bond_lengths_ex, bond_bond_angles_g,
           bond_bond_angles_ex, params, edge_index, bond_bond_index,
           edge_attr_batch, batch):
    centers_bl = jnp.asarray(np.arange(0, 2, 0.1), jnp.float32)
    centers_ba = jnp.asarray(np.arange(0, np.pi, 0.1), jnp.float32)
    atom_x = x @ params['embed_atom']['W'] + params['embed_atom']['b']
    edge_x = edge_attr @ params['embed_bond']['W'] + params['embed_bond']['b']
    edge_x = edge_x + _rbf_embed(bond_lengths_g, centers_bl, params['bl_fc']['W'], params['bl_fc']['b'])
    edge_x = edge_x + _rbf_embed(bond_lengths_ex, centers_bl, params['bl_fc']['W'], params['bl_fc']['b'])
    angle_x = _rbf_embed(bond_bond_angles_g, centers_ba, params['ba_fc']['W'], params['ba_fc']['b'])
    angle_x = angle_x + _rbf_embed(bond_bond_angles_ex, centers_ba, params['ba_fc']['W'], params['ba_fc']['b'])

    num_nodes = x.shape[0]
    num_edges = edge_attr.shape[0]
    edge_index = edge_index.astype(jnp.int32)
    bond_bond_index = bond_bond_index.astype(jnp.int32)
    ebatch3 = edge_attr_batch.astype(jnp.int32).reshape(num_edges // BLK, 1, BLK)
    nbatch3 = batch.astype(jnp.int32).reshape(num_nodes // BLK, 1, BLK)

    atom_x = _pad128(atom_x)
    edge_x = _pad128(edge_x)
    for i in range(N_LAYERS):
        edge_x = _gin_block(edge_x, bond_bond_index[0], bond_bond_index[1],
                            angle_x, ebatch3, params['bond_layers'][i], num_edges)
        atom_x = _gin_block(atom_x, edge_index[0], edge_index[1],
                            edge_x[:, :EMBED], nbatch3, params['atom_layers'][i],
                            num_nodes)

    s = _pool_stats(atom_x, nbatch3)
    return s[:, :EMBED] / jnp.clip(s[:, EMBED], 1.0)[:, None]


# speedup vs baseline: 5.0246x; 1.0001x over previous
import functools

import jax
import jax.numpy as jnp
import numpy as np
from jax import lax
from jax.experimental import pallas as pl
from jax.experimental.pallas import tpu as pltpu
from jax.experimental.pallas import tpu_sc as plsc

NUM_GRAPHS = 256
EMBED = 32
PADW = 128
N_LAYERS = 3
EPS = 1e-5
GAMMA = 10.0

BLK = 4000


_SC_MESH = plsc.VectorSubcoreMesh(core_axis_name="c", subcore_axis_name="s")
_SC_C = 400


def _sc_gather(table, idx):
    n = idx.shape[0]
    per_w = n // 32
    nchunk = per_w // _SC_C

    @functools.partial(
        pl.kernel, mesh=_SC_MESH,
        out_type=jax.ShapeDtypeStruct((n, PADW), jnp.float32),
        scratch_types=[
            pltpu.VMEM((_SC_C,), jnp.int32),
            pltpu.VMEM((_SC_C, PADW), jnp.float32),
            pltpu.SemaphoreType.DMA,
        ],
    )
    def k(table_h, idx_h, out_h, idx_v, rows_v, sem):
        wid = lax.axis_index("s") * 2 + lax.axis_index("c")
        base = wid * per_w

        def body(j, carry):
            off = base + j * _SC_C
            pltpu.sync_copy(idx_h.at[pl.ds(off, _SC_C)], idx_v)
            pltpu.async_copy(table_h.at[idx_v], rows_v, sem).wait()
            pltpu.sync_copy(rows_v, out_h.at[pl.ds(off, _SC_C)])
            return carry

        lax.fori_loop(0, nchunk, body, 0)

    return k(table, idx)


def _mlp_stats_body(x_ref, agg_ref, b_ref, w1_ref, b1_ref, w2_ref, b2_ref,
                    y_ref, s_ref):
    h = x_ref[:, :EMBED] + agg_ref[...]
    t = jnp.maximum(h @ w1_ref[...] + b1_ref[...], 0.0)
    y = t @ w2_ref[...] + b2_ref[...]
    y_ref[...] = jnp.concatenate(
        [y, jnp.zeros((BLK, PADW - EMBED), jnp.float32)], axis=1)
    ids = b_ref[0]
    oh = (ids[:, :, None] == lax.broadcasted_iota(jnp.int32, (1, 1, NUM_GRAPHS), 2))
    oh = oh.reshape(BLK, NUM_GRAPHS).astype(jnp.float32)
    yaug = jnp.concatenate([y, y * y, jnp.ones((BLK, 1), jnp.float32)], axis=1)
    part = lax.dot_general(oh, yaug, (((0,), (0,)), ((), ())),
                           preferred_element_type=jnp.float32)

    @pl.when(pl.program_id(0) == 0)
    def _():
        s_ref[...] = jnp.zeros_like(s_ref)

    s_ref[...] += part


def _mlp_stats(x, agg, batch3, p):
    m = x.shape[0]
    nb = m // BLK
    return pl.pallas_call(
        _mlp_stats_body,
        grid=(nb,),
        in_specs=[
            pl.BlockSpec((BLK, PADW), lambda i: (i, 0)),
            pl.BlockSpec((BLK, EMBED), lambda i: (i, 0)),
            pl.BlockSpec((1, 1, BLK), lambda i: (i, 0, 0)),
            pl.BlockSpec((EMBED, 2 * EMBED), lambda i: (0, 0)),
            pl.BlockSpec((1, 2 * EMBED), lambda i: (0, 0)),
            pl.BlockSpec((2 * EMBED, EMBED), lambda i: (0, 0)),
            pl.BlockSpec((1, EMBED), lambda i: (0, 0)),
        ],
        out_specs=[
            pl.BlockSpec((BLK, PADW), lambda i: (i, 0)),
            pl.BlockSpec((NUM_GRAPHS, 2 * EMBED + 1), lambda i: (0, 0)),
        ],
        out_shape=[
            jax.ShapeDtypeStruct((m, PADW), jnp.float32),
            jax.ShapeDtypeStruct((NUM_GRAPHS, 2 * EMBED + 1), jnp.float32),
        ],
    )(x, agg, batch3, p['W1'], p['b1'][None, :], p['W2'], p['b2'][None, :])


def _rowwise_body(x_ref, y_ref, b_ref, k1_ref, k2_ref, o_ref):
    ids = b_ref[0]
    oh = (ids[:, :, None] == lax.broadcasted_iota(jnp.int32, (1, 1, NUM_GRAPHS), 2))
    oh = oh.reshape(BLK, NUM_GRAPHS).astype(jnp.float32)
    k1r = oh @ k1_ref[...]
    k2r = oh @ k2_ref[...]
    o = x_ref[:, :EMBED] + jnp.maximum(y_ref[:, :EMBED] * k1r + k2r, 0.0)
    o_ref[...] = jnp.concatenate(
        [o, jnp.zeros((BLK, PADW - EMBED), jnp.float32)], axis=1)


def _rowwise(x, y, batch3, k1, k2):
    m = x.shape[0]
    nb = m // BLK
    return pl.pallas_call(
        _rowwise_body,
        grid=(nb,),
        in_specs=[
            pl.BlockSpec((BLK, PADW), lambda i: (i, 0)),
            pl.BlockSpec((BLK, PADW), lambda i: (i, 0)),
            pl.BlockSpec((1, 1, BLK), lambda i: (i, 0, 0)),
            pl.BlockSpec((NUM_GRAPHS, EMBED), lambda i: (0, 0)),
            pl.BlockSpec((NUM_GRAPHS, EMBED), lambda i: (0, 0)),
        ],
        out_specs=pl.BlockSpec((BLK, PADW), lambda i: (i, 0)),
        out_shape=jax.ShapeDtypeStruct((m, PADW), jnp.float32),
    )(x, y, batch3, k1, k2)


def _pool_body(x_ref, b_ref, s_ref):
    ids = b_ref[0]
    oh = (ids[:, :, None] == lax.broadcasted_iota(jnp.int32, (1, 1, NUM_GRAPHS), 2))
    oh = oh.reshape(BLK, NUM_GRAPHS).astype(jnp.float32)
    xaug = jnp.concatenate([x_ref[:, :EMBED], jnp.ones((BLK, 1), jnp.float32)],
                           axis=1)
    part = lax.dot_general(oh, xaug, (((0,), (0,)), ((), ())),
                           preferred_element_type=jnp.float32)

    @pl.when(pl.program_id(0) == 0)
    def _():
        s_ref[...] = jnp.zeros_like(s_ref)

    s_ref[...] += part


def _pool_stats(x, batch3):
    m = x.shape[0]
    nb = m // BLK
    return pl.pallas_call(
        _pool_body,
        grid=(nb,),
        in_specs=[
            pl.BlockSpec((BLK, PADW), lambda i: (i, 0)),
            pl.BlockSpec((1, 1, BLK), lambda i: (i, 0, 0)),
        ],
        out_specs=pl.BlockSpec((NUM_GRAPHS, EMBED + 1), lambda i: (0, 0)),
        out_shape=jax.ShapeDtypeStruct((NUM_GRAPHS, EMBED + 1), jnp.float32),
    )(x, batch3)


def _norm_coeffs(s, p):
    s1 = s[:, :EMBED]
    s2 = s[:, EMBED:2 * EMBED]
    n = jnp.clip(s[:, 2 * EMBED], 1.0)
    denom = n * EMBED
    mean_ln = s1.sum(-1) / denom
    var_ln = s2.sum(-1) / denom - mean_ln * mean_ln
    inv_ln = 1.0 / jnp.sqrt(var_ln + EPS)
    lnw, lnb = p['ln_w'], p['ln_b']
    za = inv_ln[:, None] * lnw[None, :]
    zc = lnb[None, :] - mean_ln[:, None] * za
    ey = s1 / n[:, None]
    ey2 = s2 / n[:, None]
    ez = za * ey + zc
    ez2 = za * za * ey2 + 2.0 * za * zc * ey + zc * zc
    ms, gnw, gnb = p['gn_ms'], p['gn_w'], p['gn_b']
    var2 = ez2 - 2.0 * ms[None, :] * ez * ez + (ms * ms)[None, :] * ez * ez
    rinv2 = 1.0 / jnp.sqrt(var2 + EPS)
    k1 = za * gnw[None, :] * rinv2
    k2 = (zc - ez * ms[None, :]) * gnw[None, :] * rinv2 + gnb[None, :]
    return k1, k2


def _rbf_embed(vals, centers, W, b):
    r = jnp.exp(-GAMMA * jnp.square(vals[:, None] - centers[None, :]))
    return r @ W + b


def _gin_block_pre(x, g, dst, edge_attr, batch3, p, num_nodes):
    msg = jax.nn.relu(g[:, :EMBED] + edge_attr)
    agg = jax.ops.segment_sum(msg, dst, num_segments=num_nodes)
    y, s = _mlp_stats(x, agg, batch3, p)
    k1, k2 = _norm_coeffs(s, p)
    return _rowwise(x, y, batch3, k1, k2)


def _gin_block(x, src, dst, edge_attr, batch3, p, num_nodes):
    return _gin_block_pre(x, _sc_gather(x, src), dst, edge_attr, batch3, p,
                          num_nodes)


def _pad128(x):
    return jnp.pad(x, ((0, 0), (0, PADW - x.shape[1])))


def kernel(x, edge_attr, bond_lengths_g, bond_lengths_ex, bond_bond_angles_g,
           bond_bond_angles_ex, params, edge_index, bond_bond_index,
           edge_attr_batch, batch):
    centers_bl = jnp.asarray(np.arange(0, 2, 0.1), jnp.float32)
    centers_ba = jnp.asarray(np.arange(0, np.pi, 0.1), jnp.float32)
    atom_x = x @ params['embed_atom']['W'] + params['embed_atom']['b']
    edge_x = edge_attr @ params['embed_bond']['W'] + params['embed_bond']['b']
    edge_x = edge_x + _rbf_embed(bond_lengths_g, centers_bl, params['bl_fc']['W'], params['bl_fc']['b'])
    edge_x = edge_x + _rbf_embed(bond_lengths_ex, centers_bl, params['bl_fc']['W'], params['bl_fc']['b'])
    angle_x = _rbf_embed(bond_bond_angles_g, centers_ba, params['ba_fc']['W'], params['ba_fc']['b'])
    angle_x = angle_x + _rbf_embed(bond_bond_angles_ex, centers_ba, params['ba_fc']['W'], params['ba_fc']['b'])

    num_nodes = x.shape[0]
    num_edges = edge_attr.shape[0]
    edge_index = edge_index.astype(jnp.int32)
    bond_bond_index = bond_bond_index.astype(jnp.int32)
    ebatch3 = edge_attr_batch.astype(jnp.int32).reshape(num_edges // BLK, 1, BLK)
    nbatch3 = batch.astype(jnp.int32).reshape(num_nodes // BLK, 1, BLK)

    atom_x = _pad128(atom_x)
    edge_x = _pad128(edge_x)
    for i in range(N_LAYERS):
        g_atom = _sc_gather(atom_x, edge_index[0])
        edge_x = _gin_block(edge_x, bond_bond_index[0], bond_bond_index[1],
                            angle_x, ebatch3, params['bond_layers'][i], num_edges)
        atom_x = _gin_block_pre(atom_x, g_atom, edge_index[1],
                                edge_x[:, :EMBED], nbatch3,
                                params['atom_layers'][i], num_nodes)

    s = _pool_stats(atom_x, nbatch3)
    return s[:, :EMBED] / jnp.clip(s[:, EMBED], 1.0)[:, None]


# speedup vs baseline: 5.0250x; 1.0001x over previous
import functools

import jax
import jax.numpy as jnp
import numpy as np
from jax import lax
from jax.experimental import pallas as pl
from jax.experimental.pallas import tpu as pltpu
from jax.experimental.pallas import tpu_sc as plsc

NUM_GRAPHS = 256
EMBED = 32
PADW = 128
N_LAYERS = 3
EPS = 1e-5
GAMMA = 10.0

BLK = 4000


_SC_MESH = plsc.VectorSubcoreMesh(core_axis_name="c", subcore_axis_name="s")
_SC_C = 400


def _sc_gather(table, idx):
    n = idx.shape[0]
    per_w = n // 32
    nchunk = per_w // _SC_C

    @functools.partial(
        pl.kernel, mesh=_SC_MESH,
        out_type=jax.ShapeDtypeStruct((n, PADW), jnp.float32),
        scratch_types=[
            pltpu.VMEM((_SC_C,), jnp.int32),
            pltpu.VMEM((_SC_C, PADW), jnp.float32),
            pltpu.SemaphoreType.DMA,
        ],
    )
    def k(table_h, idx_h, out_h, idx_v, rows_v, sem):
        wid = lax.axis_index("s") * 2 + lax.axis_index("c")
        base = wid * per_w

        def body(j, carry):
            off = base + j * _SC_C
            pltpu.sync_copy(idx_h.at[pl.ds(off, _SC_C)], idx_v)
            pltpu.async_copy(table_h.at[idx_v], rows_v, sem).wait()
            pltpu.sync_copy(rows_v, out_h.at[pl.ds(off, _SC_C)])
            return carry

        lax.fori_loop(0, nchunk, body, 0)

    return k(table, idx)


def _mlp_stats_body(x_ref, agg_ref, b_ref, w1_ref, b1_ref, w2_ref, b2_ref,
                    y_ref, s_ref):
    h = x_ref[:, :EMBED] + agg_ref[...]
    t = jnp.maximum(h @ w1_ref[...] + b1_ref[...], 0.0)
    y = t @ w2_ref[...] + b2_ref[...]
    y_ref[...] = jnp.concatenate(
        [y, jnp.zeros((BLK, PADW - EMBED), jnp.float32)], axis=1)
    ids = b_ref[0]
    oh = (ids[:, :, None] == lax.broadcasted_iota(jnp.int32, (1, 1, NUM_GRAPHS), 2))
    oh = oh.reshape(BLK, NUM_GRAPHS).astype(jnp.float32)
    yaug = jnp.concatenate([y, y * y, jnp.ones((BLK, 1), jnp.float32)], axis=1)
    part = lax.dot_general(oh, yaug, (((0,), (0,)), ((), ())),
                           preferred_element_type=jnp.float32)

    @pl.when(pl.program_id(0) == 0)
    def _():
        s_ref[...] = jnp.zeros_like(s_ref)

    s_ref[...] += part


def _mlp_stats(x, agg, batch3, p):
    m = x.shape[0]
    nb = m // BLK
    return pl.pallas_call(
        _mlp_stats_body,
        grid=(nb,),
        in_specs=[
            pl.BlockSpec((BLK, PADW), lambda i: (i, 0)),
            pl.BlockSpec((BLK, EMBED), lambda i: (i, 0)),
            pl.BlockSpec((1, 1, BLK), lambda i: (i, 0, 0)),
            pl.BlockSpec((EMBED, 2 * EMBED), lambda i: (0, 0)),
            pl.BlockSpec((1, 2 * EMBED), lambda i: (0, 0)),
            pl.BlockSpec((2 * EMBED, EMBED), lambda i: (0, 0)),
            pl.BlockSpec((1, EMBED), lambda i: (0, 0)),
        ],
        out_specs=[
            pl.BlockSpec((BLK, PADW), lambda i: (i, 0)),
            pl.BlockSpec((NUM_GRAPHS, 2 * EMBED + 1), lambda i: (0, 0)),
        ],
        out_shape=[
            jax.ShapeDtypeStruct((m, PADW), jnp.float32),
            jax.ShapeDtypeStruct((NUM_GRAPHS, 2 * EMBED + 1), jnp.float32),
        ],
    )(x, agg, batch3, p['W1'], p['b1'][None, :], p['W2'], p['b2'][None, :])


def _rowwise_body(x_ref, y_ref, b_ref, k1_ref, k2_ref, o_ref):
    ids = b_ref[0]
    oh = (ids[:, :, None] == lax.broadcasted_iota(jnp.int32, (1, 1, NUM_GRAPHS), 2))
    oh = oh.reshape(BLK, NUM_GRAPHS).astype(jnp.float32)
    k1r = oh @ k1_ref[...]
    k2r = oh @ k2_ref[...]
    o = x_ref[:, :EMBED] + jnp.maximum(y_ref[:, :EMBED] * k1r + k2r, 0.0)
    o_ref[...] = jnp.concatenate(
        [o, jnp.zeros((BLK, PADW - EMBED), jnp.float32)], axis=1)


def _rowwise(x, y, batch3, k1, k2):
    m = x.shape[0]
    nb = m // BLK
    return pl.pallas_call(
        _rowwise_body,
        grid=(nb,),
        in_specs=[
            pl.BlockSpec((BLK, PADW), lambda i: (i, 0)),
            pl.BlockSpec((BLK, PADW), lambda i: (i, 0)),
            pl.BlockSpec((1, 1, BLK), lambda i: (i, 0, 0)),
            pl.BlockSpec((NUM_GRAPHS, EMBED), lambda i: (0, 0)),
            pl.BlockSpec((NUM_GRAPHS, EMBED), lambda i: (0, 0)),
        ],
        out_specs=pl.BlockSpec((BLK, PADW), lambda i: (i, 0)),
        out_shape=jax.ShapeDtypeStruct((m, PADW), jnp.float32),
    )(x, y, batch3, k1, k2)


def _pool_body(x_ref, b_ref, s_ref):
    ids = b_ref[0]
    oh = (ids[:, :, None] == lax.broadcasted_iota(jnp.int32, (1, 1, NUM_GRAPHS), 2))
    oh = oh.reshape(BLK, NUM_GRAPHS).astype(jnp.float32)
    xaug = jnp.concatenate([x_ref[:, :EMBED], jnp.ones((BLK, 1), jnp.float32)],
                           axis=1)
    part = lax.dot_general(oh, xaug, (((0,), (0,)), ((), ())),
                           preferred_element_type=jnp.float32)

    @pl.when(pl.program_id(0) == 0)
    def _():
        s_ref[...] = jnp.zeros_like(s_ref)

    s_ref[...] += part


def _pool_stats(x, batch3):
    m = x.shape[0]
    nb = m // BLK
    return pl.pallas_call(
        _pool_body,
        grid=(nb,),
        in_specs=[
            pl.BlockSpec((BLK, PADW), lambda i: (i, 0)),
            pl.BlockSpec((1, 1, BLK), lambda i: (i, 0, 0)),
        ],
        out_specs=pl.BlockSpec((NUM_GRAPHS, EMBED + 1), lambda i: (0, 0)),
        out_shape=jax.ShapeDtypeStruct((NUM_GRAPHS, EMBED + 1), jnp.float32),
    )(x, batch3)


def _norm_coeffs(s, p):
    s1 = s[:, :EMBED]
    s2 = s[:, EMBED:2 * EMBED]
    n = jnp.clip(s[:, 2 * EMBED], 1.0)
    denom = n * EMBED
    mean_ln = s1.sum(-1) / denom
    var_ln = s2.sum(-1) / denom - mean_ln * mean_ln
    inv_ln = 1.0 / jnp.sqrt(var_ln + EPS)
    lnw, lnb = p['ln_w'], p['ln_b']
    za = inv_ln[:, None] * lnw[None, :]
    zc = lnb[None, :] - mean_ln[:, None] * za
    ey = s1 / n[:, None]
    ey2 = s2 / n[:, None]
    ez = za * ey + zc
    ez2 = za * za * ey2 + 2.0 * za * zc * ey + zc * zc
    ms, gnw, gnb = p['gn_ms'], p['gn_w'], p['gn_b']
    var2 = ez2 - 2.0 * ms[None, :] * ez * ez + (ms * ms)[None, :] * ez * ez
    rinv2 = 1.0 / jnp.sqrt(var2 + EPS)
    k1 = za * gnw[None, :] * rinv2
    k2 = (zc - ez * ms[None, :]) * gnw[None, :] * rinv2 + gnb[None, :]
    return k1, k2


def _rbf_embed(vals, centers, W, b):
    r = jnp.exp(-GAMMA * jnp.square(vals[:, None] - centers[None, :]))
    return r @ W + b


def _gin_block_pre(x, g, dst, edge_attr, batch3, p, num_nodes):
    msg = jax.nn.relu(g[:, :EMBED] + edge_attr)
    agg = jax.ops.segment_sum(msg, dst, num_segments=num_nodes)
    y, s = _mlp_stats(x, agg, batch3, p)
    k1, k2 = _norm_coeffs(s, p)
    return _rowwise(x, y, batch3, k1, k2)


def _gin_block(x, src, dst, edge_attr, batch3, p, num_nodes):
    return _gin_block_pre(x, _sc_gather(x, src), dst, edge_attr, batch3, p,
                          num_nodes)


def _pad128(x):
    return jnp.pad(x, ((0, 0), (0, PADW - x.shape[1])))


def kernel(x, edge_attr, bond_lengths_g, bond_lengths_ex, bond_bond_angles_g,
           bond_bond_angles_ex, params, edge_index, bond_bond_index,
           edge_attr_batch, batch):
    centers_bl = jnp.asarray(np.arange(0, 2, 0.1), jnp.float32)
    centers_ba = jnp.asarray(np.arange(0, np.pi, 0.1), jnp.float32)
    atom_x = x @ params['embed_atom']['W'] + params['embed_atom']['b']
    edge_x = edge_attr @ params['embed_bond']['W'] + params['embed_bond']['b']
    edge_x = edge_x + _rbf_embed(bond_lengths_g, centers_bl, params['bl_fc']['W'], params['bl_fc']['b'])
    edge_x = edge_x + _rbf_embed(bond_lengths_ex, centers_bl, params['bl_fc']['W'], params['bl_fc']['b'])
    angle_x = _rbf_embed(bond_bond_angles_g, centers_ba, params['ba_fc']['W'], params['ba_fc']['b'])
    angle_x = angle_x + _rbf_embed(bond_bond_angles_ex, centers_ba, params['ba_fc']['W'], params['ba_fc']['b'])

    num_nodes = x.shape[0]
    num_edges = edge_attr.shape[0]
    edge_index = edge_index.astype(jnp.int32)
    bond_bond_index = bond_bond_index.astype(jnp.int32)
    ebatch3 = edge_attr_batch.astype(jnp.int32).reshape(num_edges // BLK, 1, BLK)
    nbatch3 = batch.astype(jnp.int32).reshape(num_nodes // BLK, 1, BLK)

    atom_x = _pad128(atom_x)
    edge_x = _pad128(edge_x)
    for i in range(N_LAYERS):
        edge_x = _gin_block(edge_x, bond_bond_index[0], bond_bond_index[1],
                            angle_x, ebatch3, params['bond_layers'][i], num_edges)
        atom_x = _gin_block(atom_x, edge_index[0], edge_index[1],
                            edge_x[:, :EMBED], nbatch3,
                            params['atom_layers'][i], num_nodes)

    s = _pool_stats(atom_x, nbatch3)
    return s[:, :EMBED] / jnp.clip(s[:, EMBED], 1.0)[:, None]
